# Initial kernel scaffold; baseline (speedup 1.0000x reference)
import jax
import jax.numpy as jnp
from jax.experimental import pallas as pl


def kernel(x, edge_index, weight, self_loops, bias):
    raise NotImplementedError("write your pallas kernel here")


# speedup vs baseline: 3.2592x; 3.2592x over previous
import functools

import jax
import jax.numpy as jnp
from jax import lax
from jax.experimental import pallas as pl
from jax.experimental.pallas import tpu as pltpu
from jax.experimental.pallas import tpu_sc as plsc

NS = 16
NSC = 2
RB = 16


def _tc_body(x_ref, w_ref, s_ref, b_ref, y_ref, z_ref):
    xb = x_ref[...]
    y_ref[...] = jnp.dot(xb, w_ref[...], preferred_element_type=jnp.float32)
    zp = jnp.dot(xb, s_ref[...], preferred_element_type=jnp.float32) + b_ref[...]
    z_ref[0] = zp
    z_ref[1] = jnp.zeros_like(zp)


def _tc_prep(x, weight, self_loops, bias):
    N, D = x.shape
    R = 1000
    grid = (N // R,)
    return pl.pallas_call(
        _tc_body,
        grid=grid,
        in_specs=[
            pl.BlockSpec((R, D), lambda i: (i, 0)),
            pl.BlockSpec((D, D), lambda i: (0, 0)),
            pl.BlockSpec((D, D), lambda i: (0, 0)),
            pl.BlockSpec((1, D), lambda i: (0, 0)),
        ],
        out_specs=[
            pl.BlockSpec((R, D), lambda i: (i, 0)),
            pl.BlockSpec((2, R, D), lambda i: (0, i, 0)),
        ],
        out_shape=[
            jax.ShapeDtypeStruct((N, D), jnp.float32),
            jax.ShapeDtypeStruct((2, N, D), jnp.float32),
        ],
    )(x, weight, self_loops, bias.reshape(1, D))


def _make_sc_kernel(N, D, CH):
    rows = -(-N // (NS * 8)) * 8
    NACC = rows * NS
    mesh = plsc.VectorSubcoreMesh(core_axis_name="c", subcore_axis_name="s")

    @functools.partial(
        pl.kernel,
        out_type=jax.ShapeDtypeStruct((NSC, NACC, D), jnp.float32),
        mesh=mesh,
        scratch_types=[
            pltpu.VMEM((RB, 128), jnp.int32),
            pltpu.VMEM((RB, 128), jnp.int32),
            pltpu.VMEM((128, D), jnp.float32),
            pltpu.VMEM_SHARED((NACC, D), jnp.float32),
            pltpu.SemaphoreType.DMA,
            pltpu.SemaphoreType.DMA,
        ],
    )
    def sc_fn(y_hbm, z_hbm, src_hbm, dst_hbm, out_hbm,
              src_v, dst_v, gbuf, acc_sh, gsem, ssem):
        c = lax.axis_index("c")
        s = lax.axis_index("s")
        w = c * NS + s

        @pl.when(s == 0)
        def _():
            pltpu.sync_copy(z_hbm.at[c], acc_sh.at[pl.ds(0, N)])

        plsc.subcore_barrier()

        def group(g, carry):
            pltpu.sync_copy(src_hbm.at[w, pl.ds(g * RB, RB)], src_v)
            pltpu.sync_copy(dst_hbm.at[w, pl.ds(g * RB, RB)], dst_v)

            def body(j, carry):
                pltpu.async_copy(y_hbm.at[dst_v.at[j]], gbuf, gsem).wait()
                pltpu.async_copy(gbuf, acc_sh.at[src_v.at[j]], ssem, add=True).wait()
                return carry

            return lax.fori_loop(0, RB, body, carry)

        lax.fori_loop(0, CH // RB, group, 0)
        plsc.subcore_barrier()

        pltpu.sync_copy(
            acc_sh.at[pl.ds(s * rows, rows)],
            out_hbm.at[c, pl.ds(s * rows, rows)],
        )

    return sc_fn


def kernel(x, edge_index, weight, self_loops, bias):
    N, D = x.shape
    E = edge_index.shape[0]
    NW = NSC * NS
    EPW = -(-E // (NW * 128 * RB)) * 128 * RB
    CH = EPW // 128
    pad = EPW * NW - E

    y, z2 = _tc_prep(x, weight, self_loops, bias)

    src = edge_index[:, 0]
    dst = edge_index[:, 1]
    srcp = jnp.concatenate([src, jnp.full((pad,), N, jnp.int32)]).reshape(NW, CH, 128)
    dstp = jnp.concatenate([dst, jnp.zeros((pad,), jnp.int32)]).reshape(NW, CH, 128)

    out_sc = _make_sc_kernel(N, D, CH)(y, z2, srcp, dstp)
    return out_sc[0, :N] + out_sc[1, :N]


# speedup vs baseline: 3.6396x; 1.1167x over previous
import functools

import jax
import jax.numpy as jnp
from jax import lax
from jax.experimental import pallas as pl
from jax.experimental.pallas import tpu as pltpu
from jax.experimental.pallas import tpu_sc as plsc

NS = 16
NSC = 2
RB = 16


def _tc_body(x_ref, w_ref, s_ref, b_ref, y_ref, z_ref):
    xb = x_ref[...]
    y_ref[...] = jnp.dot(xb, w_ref[...], preferred_element_type=jnp.float32)
    zp = jnp.dot(xb, s_ref[...], preferred_element_type=jnp.float32) + b_ref[...]
    z_ref[0] = zp
    z_ref[1] = jnp.zeros_like(zp)


def _tc_prep(x, weight, self_loops, bias):
    N, D = x.shape
    R = 1000
    grid = (N // R,)
    return pl.pallas_call(
        _tc_body,
        grid=grid,
        in_specs=[
            pl.BlockSpec((R, D), lambda i: (i, 0)),
            pl.BlockSpec((D, D), lambda i: (0, 0)),
            pl.BlockSpec((D, D), lambda i: (0, 0)),
            pl.BlockSpec((1, D), lambda i: (0, 0)),
        ],
        out_specs=[
            pl.BlockSpec((R, D), lambda i: (i, 0)),
            pl.BlockSpec((2, R, D), lambda i: (0, i, 0)),
        ],
        out_shape=[
            jax.ShapeDtypeStruct((N, D), jnp.float32),
            jax.ShapeDtypeStruct((2, N, D), jnp.float32),
        ],
    )(x, weight, self_loops, bias.reshape(1, D))


def _make_sc_kernel(N, D, CH):
    rows = -(-N // (NS * 8)) * 8
    NACC = rows * NS
    mesh = plsc.VectorSubcoreMesh(core_axis_name="c", subcore_axis_name="s")

    CHG = CH // RB

    @functools.partial(
        pl.kernel,
        out_type=jax.ShapeDtypeStruct((NSC, NACC, D), jnp.float32),
        mesh=mesh,
        scratch_types=[
            pltpu.VMEM((2, RB, 128), jnp.int32),
            pltpu.VMEM((2, RB, 128), jnp.int32),
            pltpu.VMEM((2, 128, D), jnp.float32),
            pltpu.VMEM_SHARED((NACC, D), jnp.float32),
            pltpu.SemaphoreType.DMA,
            pltpu.SemaphoreType.DMA,
            pltpu.SemaphoreType.DMA,
        ],
    )
    def sc_fn(y_hbm, z_hbm, src_hbm, dst_hbm, out_hbm,
              src_v, dst_v, gbuf, acc_sh, gsem, ssem, rsem):
        c = lax.axis_index("c")
        s = lax.axis_index("s")
        w = c * NS + s

        def drain(dst, sem):
            pltpu.make_async_copy(y_hbm.at[pl.ds(0, dst.shape[0])], dst, sem).wait()

        @pl.when(s == 0)
        def _():
            pltpu.sync_copy(z_hbm.at[c], acc_sh.at[pl.ds(0, N)])

        pltpu.async_copy(src_hbm.at[w, pl.ds(0, RB)], src_v.at[0], rsem)
        pltpu.async_copy(dst_hbm.at[w, pl.ds(0, RB)], dst_v.at[0], rsem)
        plsc.subcore_barrier()

        def group(g, carry):
            p = lax.rem(g, 2)
            pltpu.make_async_copy(src_hbm.at[w, pl.ds(0, RB)], src_v.at[0], rsem).wait()
            pltpu.make_async_copy(dst_hbm.at[w, pl.ds(0, RB)], dst_v.at[0], rsem).wait()

            @pl.when(g + 1 < CHG)
            def _():
                pn = lax.rem(g + 1, 2)
                pltpu.async_copy(src_hbm.at[w, pl.ds((g + 1) * RB, RB)], src_v.at[pn], rsem)
                pltpu.async_copy(dst_hbm.at[w, pl.ds((g + 1) * RB, RB)], dst_v.at[pn], rsem)

            pltpu.async_copy(y_hbm.at[dst_v.at[p, 0]], gbuf.at[0], gsem)

            def body(j, carry):
                b = lax.rem(j, 2)
                bn = lax.rem(j + 1, 2)

                @pl.when(j > 0)
                def _():
                    drain(gbuf.at[0], ssem)

                @pl.when(j + 1 < RB)
                def _():
                    pltpu.async_copy(y_hbm.at[dst_v.at[p, j + 1]], gbuf.at[bn], gsem)

                drain(gbuf.at[0], gsem)
                pltpu.async_copy(gbuf.at[b], acc_sh.at[src_v.at[p, j]], ssem, add=True)
                return carry

            carry = lax.fori_loop(0, RB, body, carry)
            drain(gbuf.at[0], ssem)
            return carry

        lax.fori_loop(0, CHG, group, 0)
        plsc.subcore_barrier()

        pltpu.sync_copy(
            acc_sh.at[pl.ds(s * rows, rows)],
            out_hbm.at[c, pl.ds(s * rows, rows)],
        )

    return sc_fn


def kernel(x, edge_index, weight, self_loops, bias):
    N, D = x.shape
    E = edge_index.shape[0]
    NW = NSC * NS
    EPW = -(-E // (NW * 128 * RB)) * 128 * RB
    CH = EPW // 128
    pad = EPW * NW - E

    y, z2 = _tc_prep(x, weight, self_loops, bias)

    src = edge_index[:, 0]
    dst = edge_index[:, 1]
    srcp = jnp.concatenate([src, jnp.full((pad,), N, jnp.int32)]).reshape(NW, CH, 128)
    dstp = jnp.concatenate([dst, jnp.zeros((pad,), jnp.int32)]).reshape(NW, CH, 128)

    out_sc = _make_sc_kernel(N, D, CH)(y, z2, srcp, dstp)
    return out_sc[0, :N] + out_sc[1, :N]


# speedup vs baseline: 4.1727x; 1.1465x over previous
import functools

import jax
import jax.numpy as jnp
from jax import lax
from jax.experimental import pallas as pl
from jax.experimental.pallas import tpu as pltpu
from jax.experimental.pallas import tpu_sc as plsc

NS = 16
NSC = 2
RB = 16
CH0 = 128
CH1 = 32


def _tc_body(x_ref, w_ref, s_ref, b_ref, y_ref, z_ref):
    xb = x_ref[...]
    y_ref[...] = jnp.dot(xb, w_ref[...], preferred_element_type=jnp.float32)
    z_ref[...] = jnp.dot(xb, s_ref[...], preferred_element_type=jnp.float32) + b_ref[...]


def _tc_prep(x, weight, self_loops, bias):
    N, D = x.shape
    R = 1000
    grid = (N // R,)
    return pl.pallas_call(
        _tc_body,
        grid=grid,
        in_specs=[
            pl.BlockSpec((R, D), lambda i: (i, 0)),
            pl.BlockSpec((D, D), lambda i: (0, 0)),
            pl.BlockSpec((D, D), lambda i: (0, 0)),
            pl.BlockSpec((1, D), lambda i: (0, 0)),
        ],
        out_specs=[
            pl.BlockSpec((R, D), lambda i: (i, 0)),
            pl.BlockSpec((R, D), lambda i: (i, 0)),
        ],
        out_shape=[
            jax.ShapeDtypeStruct((N, D), jnp.float32),
            jax.ShapeDtypeStruct((N, D), jnp.float32),
        ],
    )(x, weight, self_loops, bias.reshape(1, D))


def _make_sc_kernel(N, D):
    rows = -(-N // (NS * 8)) * 8
    NACC = rows * NS
    full, rem = divmod(rows, 128)
    lastz = N - (NS - 1) * rows
    mesh = plsc.VectorSubcoreMesh(core_axis_name="c", subcore_axis_name="s")

    @functools.partial(
        pl.kernel,
        out_type=jax.ShapeDtypeStruct((NSC, NACC, D), jnp.float32),
        mesh=mesh,
        scratch_types=[
            pltpu.VMEM((2, RB, 128), jnp.int32),
            pltpu.VMEM((2, RB, 128), jnp.int32),
            pltpu.VMEM((2, 128, D), jnp.float32),
            pltpu.VMEM_SHARED((NACC, D), jnp.float32),
            pltpu.SemaphoreType.DMA,
            pltpu.SemaphoreType.DMA,
            pltpu.SemaphoreType.DMA,
        ],
    )
    def sc_fn(y_hbm, z_hbm, src_hbm, dst_hbm, out_hbm,
              src_v, dst_v, gbuf, acc_sh, gsem, ssem, rsem):
        c = lax.axis_index("c")
        s = lax.axis_index("s")
        coff = jnp.where(c == 0, s * CH0, NS * CH0 + s * CH1)
        chg = jnp.where(c == 0, CH0 // RB, CH1 // RB)

        def drain(dst, sem):
            pltpu.make_async_copy(y_hbm.at[pl.ds(0, dst.shape[0])], dst, sem).wait()

        @pl.when(c == 0)
        def _():
            @pl.when(s < NS - 1)
            def _():
                pltpu.sync_copy(z_hbm.at[pl.ds(s * rows, rows)],
                                acc_sh.at[pl.ds(s * rows, rows)])

            @pl.when(s == NS - 1)
            def _():
                pltpu.sync_copy(z_hbm.at[pl.ds((NS - 1) * rows, lastz)],
                                acc_sh.at[pl.ds((NS - 1) * rows, lastz)])

        @pl.when(c == 1)
        def _():
            def zrow(r, carry):
                for k in range(D // 16):
                    gbuf[0, r, pl.ds(k * 16, 16)] = jnp.zeros((16,), jnp.float32)
                return carry

            lax.fori_loop(0, 128, zrow, 0)
            for t in range(full):
                pltpu.sync_copy(gbuf.at[0],
                                acc_sh.at[pl.ds(s * rows + t * 128, 128)])
            if rem:
                pltpu.sync_copy(gbuf.at[0, pl.ds(0, rem)],
                                acc_sh.at[pl.ds(s * rows + full * 128, rem)])

        pltpu.async_copy(src_hbm.at[pl.ds(coff, RB)], src_v.at[0], rsem)
        pltpu.async_copy(dst_hbm.at[pl.ds(coff, RB)], dst_v.at[0], rsem)
        plsc.subcore_barrier()

        def group(g, carry):
            p = lax.rem(g, 2)
            pltpu.make_async_copy(src_hbm.at[pl.ds(0, RB)], src_v.at[0], rsem).wait()
            pltpu.make_async_copy(dst_hbm.at[pl.ds(0, RB)], dst_v.at[0], rsem).wait()

            @pl.when(g + 1 < chg)
            def _():
                pn = lax.rem(g + 1, 2)
                pltpu.async_copy(src_hbm.at[pl.ds(coff + (g + 1) * RB, RB)],
                                 src_v.at[pn], rsem)
                pltpu.async_copy(dst_hbm.at[pl.ds(coff + (g + 1) * RB, RB)],
                                 dst_v.at[pn], rsem)

            pltpu.async_copy(y_hbm.at[dst_v.at[p, 0]], gbuf.at[0], gsem)

            def body(j, carry):
                b = lax.rem(j, 2)
                bn = lax.rem(j + 1, 2)

                @pl.when(j > 0)
                def _():
                    drain(gbuf.at[0], ssem)

                @pl.when(j + 1 < RB)
                def _():
                    pltpu.async_copy(y_hbm.at[dst_v.at[p, j + 1]], gbuf.at[bn], gsem)

                drain(gbuf.at[0], gsem)
                pltpu.async_copy(gbuf.at[b], acc_sh.at[src_v.at[p, j]], ssem, add=True)
                return carry

            carry = lax.fori_loop(0, RB, body, carry)
            drain(gbuf.at[0], ssem)
            return carry

        lax.fori_loop(0, chg, group, 0)
        plsc.subcore_barrier()

        pltpu.sync_copy(
            acc_sh.at[pl.ds(s * rows, rows)],
            out_hbm.at[c, pl.ds(s * rows, rows)],
        )

    return sc_fn


def kernel(x, edge_index, weight, self_loops, bias):
    N, D = x.shape
    E = edge_index.shape[0]
    CT = NS * (CH0 + CH1)
    EP = CT * 128
    assert EP >= E
    pad = EP - E

    y, z = _tc_prep(x, weight, self_loops, bias)

    src = edge_index[:, 0]
    dst = edge_index[:, 1]
    srcp = jnp.concatenate([src, jnp.full((pad,), N, jnp.int32)]).reshape(CT, 128)
    dstp = jnp.concatenate([dst, jnp.zeros((pad,), jnp.int32)]).reshape(CT, 128)

    out_sc = _make_sc_kernel(N, D)(y, z, srcp, dstp)
    return out_sc[0, :N] + out_sc[1, :N]
